# baseline (speedup 1.0000x reference)
import functools

import jax
import jax.numpy as jnp
from jax.experimental import pallas as pl
from jax.experimental.pallas import tpu as pltpu
from jax.experimental.pallas import tpu_sc as plsc

_N = 5000
_B = 128
_NP = 5120
_NB = _NP // _B
_T = 0.3

_NW = 32
_RPW = _NP // _NW
_CH = 80
_NCH = _NP // _CH


def _sc_gather(boxes, scores, opad2):
    boxes_flat = boxes.reshape(-1)
    mesh = plsc.VectorSubcoreMesh(core_axis_name="c", subcore_axis_name="s")

    @functools.partial(
        pl.kernel,
        out_type=(jax.ShapeDtypeStruct((_NCH, _CH), jnp.float32),
                  jax.ShapeDtypeStruct((4, _NCH, _CH), jnp.float32)),
        mesh=mesh,
        scratch_types=[
            pltpu.VMEM((2, _CH), jnp.int32),
            pltpu.VMEM((2, _CH), jnp.int32),
            pltpu.VMEM((2, _CH), jnp.float32),
            pltpu.VMEM((2, _CH), jnp.float32),
            pltpu.SemaphoreType.DMA,
        ],
    )
    def k(boxesf_h, scores_h, opad_h, s_h, bt_h,
          idxv, idxs, sv, btv, sem):
        w = jax.lax.axis_index("s") * 2 + jax.lax.axis_index("c")
        r0 = w * 2
        pltpu.sync_copy(opad_h.at[pl.ds(r0, 2)], idxv)
        for t in range(2):
            pltpu.async_copy(scores_h.at[idxv.at[t]], sv.at[t], sem).wait()
        pltpu.sync_copy(sv, s_h.at[pl.ds(r0, 2)])
        for k4 in range(4):
            for t in range(2):
                for u in range(_CH // 16):
                    idxs[t, pl.ds(16 * u, 16)] = (
                        idxv[t, pl.ds(16 * u, 16)] * 4 + k4)
            for t in range(2):
                pltpu.async_copy(boxesf_h.at[idxs.at[t]],
                                 btv.at[t], sem).wait()
            pltpu.sync_copy(btv, bt_h.at[k4, pl.ds(r0, 2)])

    return k(boxes_flat, scores, opad2)


def _iou_cr(c, r, ac, ar):
    xx1 = jnp.maximum(c[0], r[0])
    yy1 = jnp.maximum(c[1], r[1])
    xx2 = jnp.minimum(c[2], r[2])
    yy2 = jnp.minimum(c[3], r[3])
    w = jnp.maximum(0.0, xx2 - xx1)
    h = jnp.maximum(0.0, yy2 - yy1)
    inter = w * h
    return inter / (ac + ar - inter + 1e-6)


def _block_fixpoint(k0, m):
    def fcond(st):
        return st[1]

    def fbody(st):
        k, _ = st
        cnt = jax.lax.dot_general(
            k.astype(jnp.bfloat16), m, (((1,), (0,)), ((), ())),
            preferred_element_type=jnp.float32)
        knew = k0 * jnp.where(cnt > 0.0, 0.0, 1.0)
        return (knew, jnp.any(knew != k))

    kfin, _ = jax.lax.while_loop(fcond, fbody, (k0, True))
    return kfin


def _nms_body(tb_ref, sp_ref, bt_ref, out_ref):
    rows2 = jax.lax.broadcasted_iota(jnp.int32, (_B, _B), 0)
    cols2 = jax.lax.broadcasted_iota(jnp.int32, (_B, _B), 1)
    keep = jnp.ones((1, _NP), jnp.float32)
    kcols = []
    for i in range(_NB):
        lo, hi = i * _B, (i + 1) * _B
        c = tuple(tb_ref[lo:hi, k:k + 1] for k in range(4))
        r = tuple(bt_ref[k:k + 1, lo:hi] for k in range(4))
        ac = (c[2] - c[0]) * (c[3] - c[1])
        ar = (r[2] - r[0]) * (r[3] - r[1])
        iou = _iou_cr(c, r, ac, ar)
        m = jnp.where((iou > _T) & (rows2 < cols2),
                      1.0, 0.0).astype(jnp.bfloat16)
        k0 = jax.lax.slice(keep, (0, lo), (1, hi))
        kfin = _block_fixpoint(k0, m)
        kcols.append(jnp.max(
            jnp.where(rows2 == cols2, jnp.broadcast_to(kfin, (_B, _B)), 0.0),
            axis=1, keepdims=True))
        if hi < _NP:
            rr = tuple(bt_ref[k:k + 1, hi:_NP] for k in range(4))
            arr = (rr[2] - rr[0]) * (rr[3] - rr[1])
            iou_r = _iou_cr(c, rr, ac, arr)
            sr = jnp.where(iou_r > _T, 1.0, 0.0).astype(jnp.bfloat16)
            cnt = jax.lax.dot_general(
                kfin.astype(jnp.bfloat16), sr, (((1,), (0,)), ((), ())),
                preferred_element_type=jnp.float32)
            rest = (jax.lax.slice(keep, (0, hi), (1, _NP))
                    * jnp.where(cnt > 0.0, 0.0, 1.0))
            keep = jnp.concatenate(
                [jnp.zeros((1, hi), jnp.float32), rest], axis=1)
    kcol_full = jnp.concatenate(kcols, axis=0)
    out_ref[:, 0:4] = tb_ref[...] * kcol_full
    out_ref[:, 4:5] = sp_ref[...] * kcol_full


def _nms_pallas(tb4, sp, bt):
    return pl.pallas_call(
        _nms_body,
        out_shape=jax.ShapeDtypeStruct((_NP, 8), jnp.float32),
    )(tb4, sp, bt)


def kernel(boxes, scores):
    order = jnp.argsort(-scores).astype(jnp.int32)
    opad2 = jnp.concatenate(
        [order, jnp.zeros((_NP - _N,), jnp.int32)]).reshape(_NCH, _CH)
    s2, bt3 = _sc_gather(boxes, scores, opad2)
    bt = bt3.reshape(4, _NP)
    out = _nms_pallas(bt.T, s2.reshape(_NP, 1), bt)
    return out[:_N, :5]
